# Initial kernel scaffold; baseline (speedup 1.0000x reference)
import jax
import jax.numpy as jnp
from jax.experimental import pallas as pl


def kernel(feat, cu_seqlens):
    raise NotImplementedError("write your pallas kernel here")


# speedup vs baseline: 1.3701x; 1.3701x over previous
import jax
import jax.numpy as jnp
from jax import lax
from jax.experimental import pallas as pl
from jax.experimental.pallas import tpu as pltpu
from jax.experimental.pallas import tpu_sc as plsc

_B = 16
_MAX_LEN = 2048
_D = 256
_NC = 2
_NS = 16
_NW = _NC * _NS
_RPW = (_B * _MAX_LEN) // _NW
_HALF = _MAX_LEN // 2
_CHUNK = 128
_BITS = (64, 32, 16, 8, 4, 2, 1)


def _pad_body(feat_hbm, cu_hbm, zeros_hbm, out_hbm, cu_v, buf_v, zbuf_v):
    w = lax.axis_index("s") * _NC + lax.axis_index("c")
    b = w // 2
    p0 = (w % 2) * _HALF
    obase = w * _RPW

    pltpu.sync_copy(cu_hbm, cu_v)
    pltpu.sync_copy(zeros_hbm, zbuf_v)

    cu_pair = cu_v[pl.ds(b, 16)]
    start = cu_pair[0] + p0
    n = jnp.clip(cu_pair[1] - start, 0, _RPW)

    nfull = n // _CHUNK

    def _copy_chunk(i, carry):
        pltpu.sync_copy(feat_hbm.at[pl.ds(start + i * _CHUNK, _CHUNK)], buf_v)
        pltpu.sync_copy(buf_v, out_hbm.at[pl.ds(obase + i * _CHUNK, _CHUNK)])
        return carry

    lax.fori_loop(0, nfull, _copy_chunk, 0)

    off = nfull * _CHUNK
    rem = n - off
    for bit in _BITS:
        hi_mask = (_CHUNK - 1) ^ (2 * bit - 1)

        @pl.when((rem & bit) != 0)
        def _(bit=bit, hi_mask=hi_mask):
            o = off + (rem & hi_mask)
            pltpu.sync_copy(feat_hbm.at[pl.ds(start + o, bit)],
                            buf_v.at[pl.ds(0, bit)])
            pltpu.sync_copy(buf_v.at[pl.ds(0, bit)],
                            out_hbm.at[pl.ds(obase + o, bit)])

    z = _RPW - n
    zfull = z // _CHUNK

    def _zero_chunk(i, carry):
        pltpu.sync_copy(zbuf_v, out_hbm.at[pl.ds(obase + n + i * _CHUNK, _CHUNK)])
        return carry

    lax.fori_loop(0, zfull, _zero_chunk, 0)

    zoff = n + zfull * _CHUNK
    zrem = z - zfull * _CHUNK
    for bit in _BITS:
        hi_mask = (_CHUNK - 1) ^ (2 * bit - 1)

        @pl.when((zrem & bit) != 0)
        def _(bit=bit, hi_mask=hi_mask):
            o = zoff + (zrem & hi_mask)
            pltpu.sync_copy(zbuf_v.at[pl.ds(0, bit)],
                            out_hbm.at[pl.ds(obase + o, bit)])


def kernel(feat, cu_seqlens):
    cu_pad = jnp.zeros((32,), jnp.int32).at[:_B + 1].set(
        cu_seqlens.astype(jnp.int32))
    zchunk = jnp.zeros((_CHUNK, _D), jnp.float32)
    fn = pl.kernel(
        _pad_body,
        mesh=plsc.VectorSubcoreMesh(core_axis_name="c", subcore_axis_name="s"),
        compiler_params=pltpu.CompilerParams(use_tc_tiling_on_sc=False),
        out_type=jax.ShapeDtypeStruct((_B * _MAX_LEN, _D), jnp.float32),
        scratch_types=[
            pltpu.VMEM((32,), jnp.int32),
            pltpu.VMEM((_CHUNK, _D), jnp.float32),
            pltpu.VMEM((_CHUNK, _D), jnp.float32),
        ],
    )
    out = fn(feat, cu_pad, zchunk)
    return out.reshape(_B, _MAX_LEN, _D)


# speedup vs baseline: 1.3936x; 1.0171x over previous
import jax
import jax.numpy as jnp
from jax import lax
from jax.experimental import pallas as pl
from jax.experimental.pallas import tpu as pltpu
from jax.experimental.pallas import tpu_sc as plsc

_B = 16
_MAX_LEN = 2048
_D = 256
_NC = 2
_NS = 16
_NW = _NC * _NS
_RPW = (_B * _MAX_LEN) // _NW
_HALF = _MAX_LEN // 2
_CHUNK = 128
_BITS = (64, 32, 16, 8, 4, 2, 1)


def _pad_body(feat_hbm, cu_hbm, zeros_hbm, out_hbm,
              cu_v, buf0, buf1, zbuf,
              sem_i0, sem_i1, sem_o0, sem_o1, sem_z, sem_zw):
    w = lax.axis_index("s") * _NC + lax.axis_index("c")
    b = w // 2
    p0 = (w % 2) * _HALF

    pltpu.sync_copy(cu_hbm, cu_v)
    zh = pltpu.async_copy(zeros_hbm, zbuf, sem_z)

    cu_pair = cu_v[pl.ds(b, 16)]
    start = cu_pair[0] + p0
    n = jnp.clip(cu_pair[1] - start, 0, _RPW)
    nfull = n // _CHUNK

    def _src(i):
        return feat_hbm.at[pl.ds(start + i * _CHUNK, _CHUNK)]

    def _dst(i):
        return out_hbm.at[b, pl.ds(p0 + i * _CHUNK, _CHUNK)]

    @pl.when(nfull > 0)
    def _():
        pltpu.async_copy(_src(0), buf0, sem_i0)

    @pl.when(nfull > 1)
    def _():
        pltpu.async_copy(_src(1), buf1, sem_i1)

    zh.wait()
    z = _RPW - n
    zfull = z // _CHUNK

    def _zero_fire(i, carry):
        pltpu.async_copy(
            zbuf, out_hbm.at[b, pl.ds(p0 + n + i * _CHUNK, _CHUNK)], sem_zw)
        return carry

    lax.fori_loop(0, zfull, _zero_fire, 0)

    zoff = n + zfull * _CHUNK
    zrem = z - zfull * _CHUNK
    for bit in _BITS:
        hi_mask = (_CHUNK - 1) ^ (2 * bit - 1)

        @pl.when((zrem & bit) != 0)
        def _(bit=bit, hi_mask=hi_mask):
            o = zoff + (zrem & hi_mask)
            pltpu.async_copy(zbuf.at[pl.ds(0, bit)],
                             out_hbm.at[b, pl.ds(p0 + o, bit)], sem_zw)

    def _pipe(k, carry):
        i = 2 * k
        pltpu.make_async_copy(_src(i), buf0, sem_i0).wait()
        pltpu.async_copy(buf0, _dst(i), sem_o0)

        @pl.when(i + 1 < nfull)
        def _():
            pltpu.make_async_copy(_src(i + 1), buf1, sem_i1).wait()
            pltpu.async_copy(buf1, _dst(i + 1), sem_o1)

        @pl.when(i + 2 < nfull)
        def _():
            pltpu.make_async_copy(buf0, _dst(i), sem_o0).wait()
            pltpu.async_copy(_src(i + 2), buf0, sem_i0)

        @pl.when(i + 3 < nfull)
        def _():
            pltpu.make_async_copy(buf1, _dst(i + 1), sem_o1).wait()
            pltpu.async_copy(_src(i + 3), buf1, sem_i1)

        return carry

    lax.fori_loop(0, (nfull + 1) // 2, _pipe, 0)

    @pl.when((nfull >= 2) & ((nfull - 2) % 2 == 0))
    def _():
        pltpu.make_async_copy(buf0, _dst(0), sem_o0).wait()

    @pl.when((nfull >= 2) & ((nfull - 2) % 2 == 1))
    def _():
        pltpu.make_async_copy(buf1, _dst(0), sem_o1).wait()

    @pl.when((nfull >= 1) & ((nfull - 1) % 2 == 0))
    def _():
        pltpu.make_async_copy(buf0, _dst(0), sem_o0).wait()

    @pl.when((nfull >= 1) & ((nfull - 1) % 2 == 1))
    def _():
        pltpu.make_async_copy(buf1, _dst(0), sem_o1).wait()

    off = nfull * _CHUNK
    rem = n - off
    for bit in _BITS:
        hi_mask = (_CHUNK - 1) ^ (2 * bit - 1)

        @pl.when((rem & bit) != 0)
        def _(bit=bit, hi_mask=hi_mask):
            o = off + (rem & hi_mask)
            pltpu.sync_copy(feat_hbm.at[pl.ds(start + o, bit)],
                            buf0.at[pl.ds(0, bit)])
            pltpu.sync_copy(buf0.at[pl.ds(0, bit)],
                            out_hbm.at[b, pl.ds(p0 + o, bit)])

    def _zero_drain(i, carry):
        pltpu.make_async_copy(zbuf, out_hbm.at[b, pl.ds(p0, _CHUNK)],
                              sem_zw).wait()
        return carry

    lax.fori_loop(0, zfull, _zero_drain, 0)
    for bit in _BITS:
        @pl.when((zrem & bit) != 0)
        def _(bit=bit):
            pltpu.make_async_copy(zbuf.at[pl.ds(0, bit)],
                                  out_hbm.at[b, pl.ds(p0, bit)], sem_zw).wait()


def kernel(feat, cu_seqlens):
    cu_pad = jnp.zeros((32,), jnp.int32).at[:_B + 1].set(
        cu_seqlens.astype(jnp.int32))
    zchunk = jnp.zeros((_CHUNK, _D), jnp.float32)
    fn = pl.kernel(
        _pad_body,
        mesh=plsc.VectorSubcoreMesh(core_axis_name="c", subcore_axis_name="s"),
        compiler_params=pltpu.CompilerParams(use_tc_tiling_on_sc=False),
        out_type=jax.ShapeDtypeStruct((_B, _MAX_LEN, _D), jnp.float32),
        scratch_types=[
            pltpu.VMEM((32,), jnp.int32),
            pltpu.VMEM((_CHUNK, _D), jnp.float32),
            pltpu.VMEM((_CHUNK, _D), jnp.float32),
            pltpu.VMEM((_CHUNK, _D), jnp.float32),
            pltpu.SemaphoreType.DMA,
            pltpu.SemaphoreType.DMA,
            pltpu.SemaphoreType.DMA,
            pltpu.SemaphoreType.DMA,
            pltpu.SemaphoreType.DMA,
            pltpu.SemaphoreType.DMA,
        ],
    )
    return fn(feat, cu_pad, zchunk)


# speedup vs baseline: 2.1104x; 1.5144x over previous
import jax
import jax.numpy as jnp
from jax import lax
from jax.experimental import pallas as pl
from jax.experimental.pallas import tpu as pltpu
from jax.experimental.pallas import tpu_sc as plsc

_B = 16
_MAX_LEN = 2048
_D = 256
_TOTAL = 16384
_NC = 2
_NS = 16
_NW = _NC * _NS
_RPW = (_B * _MAX_LEN) // _NW
_GPW = _RPW // 8
_HALF = _MAX_LEN // 2
_CHUNK = 128
_CG = _CHUNK // 8
_BITS = (64, 32, 16, 8)
_ZG = 16
_LANES = 16


def _pad_body(feat_hbm, cu_hbm, zeros_hbm, out_hbm,
              cu_v, buf0, buf1, zbuf, gstage,
              sem_i0, sem_i1, sem_o0, sem_o1, sem_z, sem_zw):
    w = lax.axis_index("s") * _NC + lax.axis_index("c")
    b = w // 2
    g0w = (w % 2) * _GPW
    p0 = (w % 2) * _HALF

    pltpu.sync_copy(cu_hbm, cu_v)
    zh = pltpu.async_copy(zeros_hbm, zbuf, sem_z)

    cu_pair = cu_v[pl.ds(b, 16)]
    start = cu_pair[0] + p0
    n = jnp.clip(cu_pair[1] - start, 0, _RPW)
    n8 = n & ~7
    r = n - n8
    nfull = n8 // _CHUNK

    def _read_chunk(i, buf, sem):
        for gg in range(_CG):
            pltpu.async_copy(
                feat_hbm.at[pl.ds(start + i * _CHUNK + 8 * gg, 8)],
                buf.at[gg], sem)

    def _wait_read(buf, sem):
        for gg in range(_CG):
            pltpu.make_async_copy(feat_hbm.at[pl.ds(0, 8)], buf.at[gg],
                                  sem).wait()

    def _write_chunk(i, buf, sem):
        for c in range(2):
            pltpu.async_copy(
                buf.at[:, :, pl.ds(128 * c, 128)],
                out_hbm.at[b, pl.ds(g0w + i * _CG, _CG), c], sem)

    def _wait_write(buf, sem):
        for c in range(2):
            pltpu.make_async_copy(
                buf.at[:, :, pl.ds(128 * c, 128)],
                out_hbm.at[b, pl.ds(g0w, _CG), c], sem).wait()

    @pl.when(nfull > 0)
    def _():
        _read_chunk(0, buf0, sem_i0)

    @pl.when(nfull > 1)
    def _():
        _read_chunk(1, buf1, sem_i1)

    zh.wait()
    zg0 = n8 // 8 + jnp.where(r != 0, 1, 0)
    zg = _GPW - zg0
    zfull = zg // _ZG

    def _zero_fire(i, carry):
        pltpu.async_copy(
            zbuf, out_hbm.at[b, pl.ds(g0w + zg0 + i * _ZG, _ZG)], sem_zw)
        return carry

    lax.fori_loop(0, zfull, _zero_fire, 0)

    zoff = zg0 + zfull * _ZG
    zrem = zg - zfull * _ZG
    for gbit in (8, 4, 2, 1):
        hi_mask = (_ZG - 1) ^ (2 * gbit - 1)

        @pl.when((zrem & gbit) != 0)
        def _(gbit=gbit, hi_mask=hi_mask):
            o = zoff + (zrem & hi_mask)
            pltpu.async_copy(zbuf.at[pl.ds(0, gbit)],
                             out_hbm.at[b, pl.ds(g0w + o, gbit)], sem_zw)

    def _pipe(k, carry):
        i = 2 * k
        _wait_read(buf0, sem_i0)
        _write_chunk(i, buf0, sem_o0)

        @pl.when(i + 1 < nfull)
        def _():
            _wait_read(buf1, sem_i1)
            _write_chunk(i + 1, buf1, sem_o1)

        @pl.when(i + 2 < nfull)
        def _():
            _wait_write(buf0, sem_o0)
            _read_chunk(i + 2, buf0, sem_i0)

        @pl.when(i + 3 < nfull)
        def _():
            _wait_write(buf1, sem_o1)
            _read_chunk(i + 3, buf1, sem_i1)

        return carry

    lax.fori_loop(0, (nfull + 1) // 2, _pipe, 0)

    @pl.when((nfull >= 2) & ((nfull - 2) % 2 == 0))
    def _():
        _wait_write(buf0, sem_o0)

    @pl.when((nfull >= 2) & ((nfull - 2) % 2 == 1))
    def _():
        _wait_write(buf1, sem_o1)

    @pl.when((nfull >= 1) & ((nfull - 1) % 2 == 0))
    def _():
        _wait_write(buf0, sem_o0)

    @pl.when((nfull >= 1) & ((nfull - 1) % 2 == 1))
    def _():
        _wait_write(buf1, sem_o1)

    off = nfull * _CHUNK
    rem = n8 - off
    for bit in _BITS:
        gbit = bit // 8
        hi_mask = (_CHUNK - 1) ^ (2 * bit - 1)

        @pl.when((rem & bit) != 0)
        def _(bit=bit, gbit=gbit, hi_mask=hi_mask):
            o = off + (rem & hi_mask)
            for gg in range(gbit):
                pltpu.async_copy(feat_hbm.at[pl.ds(start + o + 8 * gg, 8)],
                                 buf0.at[gg], sem_i0)
            for gg in range(gbit):
                pltpu.make_async_copy(feat_hbm.at[pl.ds(0, 8)], buf0.at[gg],
                                      sem_i0).wait()
            for c in range(2):
                pltpu.async_copy(
                    buf0.at[pl.ds(0, gbit), :, pl.ds(128 * c, 128)],
                    out_hbm.at[b, pl.ds(g0w + o // 8, gbit), c], sem_o0)
            for c in range(2):
                pltpu.make_async_copy(
                    buf0.at[pl.ds(0, gbit), :, pl.ds(128 * c, 128)],
                    out_hbm.at[b, pl.ds(g0w, gbit), c], sem_o0).wait()

    @pl.when(r != 0)
    def _():
        rs = jnp.minimum(start + n8, _TOTAL - 8)
        sh = start + n8 - rs
        pltpu.sync_copy(feat_hbm.at[pl.ds(rs, 8)], buf0.at[0])
        zvec = jnp.zeros((_LANES,), jnp.float32)
        for c in range(2):
            for i in range(8):
                li = jnp.where(i < r, sh + i, 0)
                for k in range(128 // _LANES):
                    vec = buf0[0, li, pl.ds(128 * c + k * _LANES, _LANES)]
                    gstage[c, i, pl.ds(k * _LANES, _LANES)] = jnp.where(
                        i < r, vec, zvec)
        pltpu.sync_copy(gstage, out_hbm.at[b, g0w + n8 // 8])

    def _zero_drain(i, carry):
        pltpu.make_async_copy(zbuf, out_hbm.at[b, pl.ds(g0w, _ZG)],
                              sem_zw).wait()
        return carry

    lax.fori_loop(0, zfull, _zero_drain, 0)
    for gbit in (8, 4, 2, 1):
        @pl.when((zrem & gbit) != 0)
        def _(gbit=gbit):
            pltpu.make_async_copy(zbuf.at[pl.ds(0, gbit)],
                                  out_hbm.at[b, pl.ds(g0w, gbit)],
                                  sem_zw).wait()


def kernel(feat, cu_seqlens):
    cu_pad = jnp.zeros((32,), jnp.int32).at[:_B + 1].set(
        cu_seqlens.astype(jnp.int32))
    zblock = jnp.zeros((_ZG, 2, 8, 128), jnp.float32)
    fn = pl.kernel(
        _pad_body,
        mesh=plsc.VectorSubcoreMesh(core_axis_name="c", subcore_axis_name="s"),
        compiler_params=pltpu.CompilerParams(use_tc_tiling_on_sc=False),
        out_type=jax.ShapeDtypeStruct((_B, _MAX_LEN // 8, 2, 8, 128),
                                      jnp.float32),
        scratch_types=[
            pltpu.VMEM((32,), jnp.int32),
            pltpu.VMEM((_CG, 8, _D), jnp.float32),
            pltpu.VMEM((_CG, 8, _D), jnp.float32),
            pltpu.VMEM((_ZG, 2, 8, 128), jnp.float32),
            pltpu.VMEM((2, 8, 128), jnp.float32),
            pltpu.SemaphoreType.DMA,
            pltpu.SemaphoreType.DMA,
            pltpu.SemaphoreType.DMA,
            pltpu.SemaphoreType.DMA,
            pltpu.SemaphoreType.DMA,
            pltpu.SemaphoreType.DMA,
        ],
    )
    out5 = fn(feat, cu_pad, zblock)
    return (out5.transpose(0, 1, 3, 2, 4)
            .reshape(_B, _MAX_LEN, _D))


# speedup vs baseline: 2.3576x; 1.1172x over previous
import jax
import jax.numpy as jnp
from jax import lax
from jax.experimental import pallas as pl
from jax.experimental.pallas import tpu as pltpu
from jax.experimental.pallas import tpu_sc as plsc

_B = 16
_MAX_LEN = 2048
_D = 256
_TOTAL = 16384
_NGRP = _TOTAL // 8
_NC = 2
_NS = 16
_NW = _NC * _NS
_RPW = (_B * _MAX_LEN) // _NW
_GPW = _RPW // 8
_HALF = _MAX_LEN // 2
_CHUNK = 128
_CG = _CHUNK // 8
_BITS = (64, 32, 16, 8)
_ZG = 16
_LANES = 16


def _pad_body(feat_hbm, cu_hbm, zeros_hbm, out_hbm,
              cu_v, buf0, buf1, zbuf, gstage,
              sem_i0, sem_i1, sem_o0, sem_o1, sem_z, sem_zw):
    w = lax.axis_index("s") * _NC + lax.axis_index("c")
    b = w // 2
    g0w = (w % 2) * _GPW
    p0 = (w % 2) * _HALF

    pltpu.sync_copy(cu_hbm, cu_v)
    zh = pltpu.async_copy(zeros_hbm, zbuf, sem_z)

    cu_pair = cu_v[pl.ds(b, 16)]
    start = cu_pair[0] + p0
    n = jnp.clip(cu_pair[1] - start, 0, _RPW)
    n8 = n & ~7
    r = n - n8
    s = start & 7
    nfull = n8 // _CHUNK

    def _read_span(o, ngrp, buf, sem):
        gf = (start + o) >> 3
        for gg in range(ngrp):
            for c in range(2):
                pltpu.async_copy(feat_hbm.at[gf + gg, c],
                                 buf.at[pl.ds(8 * gg, 8), pl.ds(128 * c, 128)],
                                 sem)

        @pl.when(s != 0)
        def _():
            for c in range(2):
                pltpu.async_copy(
                    feat_hbm.at[gf + ngrp, c],
                    buf.at[pl.ds(8 * ngrp, 8), pl.ds(128 * c, 128)], sem)

    def _wait_read(o, ngrp, buf, sem):
        for gg in range(ngrp):
            for c in range(2):
                pltpu.make_async_copy(
                    feat_hbm.at[0, c],
                    buf.at[pl.ds(8 * gg, 8), pl.ds(128 * c, 128)], sem).wait()

        @pl.when(s != 0)
        def _():
            for c in range(2):
                pltpu.make_async_copy(
                    feat_hbm.at[0, c],
                    buf.at[pl.ds(8 * ngrp, 8), pl.ds(128 * c, 128)],
                    sem).wait()

    def _write_span(o, ngrp, buf, sem):
        for gg in range(ngrp):
            for c in range(2):
                pltpu.async_copy(
                    buf.at[pl.ds(s + 8 * gg, 8), pl.ds(128 * c, 128)],
                    out_hbm.at[b, g0w + o // 8 + gg, c], sem)

    def _wait_write(ngrp, buf, sem):
        for gg in range(ngrp):
            for c in range(2):
                pltpu.make_async_copy(
                    buf.at[pl.ds(s + 8 * gg, 8), pl.ds(128 * c, 128)],
                    out_hbm.at[b, g0w, c], sem).wait()

    @pl.when(nfull > 0)
    def _():
        _read_span(0, _CG, buf0, sem_i0)

    @pl.when(nfull > 1)
    def _():
        _read_span(_CHUNK, _CG, buf1, sem_i1)

    zh.wait()
    zg0 = n8 // 8 + jnp.where(r != 0, 1, 0)
    zg = _GPW - zg0
    zfull = zg // _ZG

    def _zero_fire(i, carry):
        pltpu.async_copy(
            zbuf, out_hbm.at[b, pl.ds(g0w + zg0 + i * _ZG, _ZG)], sem_zw)
        return carry

    lax.fori_loop(0, zfull, _zero_fire, 0)

    zoff = zg0 + zfull * _ZG
    zrem = zg - zfull * _ZG
    for gbit in (8, 4, 2, 1):
        hi_mask = (_ZG - 1) ^ (2 * gbit - 1)

        @pl.when((zrem & gbit) != 0)
        def _(gbit=gbit, hi_mask=hi_mask):
            o = zoff + (zrem & hi_mask)
            pltpu.async_copy(zbuf.at[pl.ds(0, gbit)],
                             out_hbm.at[b, pl.ds(g0w + o, gbit)], sem_zw)

    def _pipe(k, carry):
        i = 2 * k
        _wait_read(i * _CHUNK, _CG, buf0, sem_i0)
        _write_span(i * _CHUNK, _CG, buf0, sem_o0)

        @pl.when(i + 1 < nfull)
        def _():
            _wait_read((i + 1) * _CHUNK, _CG, buf1, sem_i1)
            _write_span((i + 1) * _CHUNK, _CG, buf1, sem_o1)

        @pl.when(i + 2 < nfull)
        def _():
            _wait_write(_CG, buf0, sem_o0)
            _read_span((i + 2) * _CHUNK, _CG, buf0, sem_i0)

        @pl.when(i + 3 < nfull)
        def _():
            _wait_write(_CG, buf1, sem_o1)
            _read_span((i + 3) * _CHUNK, _CG, buf1, sem_i1)

        return carry

    lax.fori_loop(0, (nfull + 1) // 2, _pipe, 0)

    @pl.when((nfull >= 2) & ((nfull - 2) % 2 == 0))
    def _():
        _wait_write(_CG, buf0, sem_o0)

    @pl.when((nfull >= 2) & ((nfull - 2) % 2 == 1))
    def _():
        _wait_write(_CG, buf1, sem_o1)

    @pl.when((nfull >= 1) & ((nfull - 1) % 2 == 0))
    def _():
        _wait_write(_CG, buf0, sem_o0)

    @pl.when((nfull >= 1) & ((nfull - 1) % 2 == 1))
    def _():
        _wait_write(_CG, buf1, sem_o1)

    off = nfull * _CHUNK
    rem = n8 - off
    for bit in _BITS:
        gbit = bit // 8
        hi_mask = (_CHUNK - 1) ^ (2 * bit - 1)

        @pl.when((rem & bit) != 0)
        def _(bit=bit, gbit=gbit, hi_mask=hi_mask):
            o = off + (rem & hi_mask)
            _read_span(o, gbit, buf0, sem_i0)
            _wait_read(o, gbit, buf0, sem_i0)
            _write_span(o, gbit, buf0, sem_o0)
            _wait_write(gbit, buf0, sem_o0)

    @pl.when(r != 0)
    def _():
        gB = (start + n8) >> 3
        sB = (start + n8) & 7
        for c in range(2):
            pltpu.async_copy(feat_hbm.at[gB, c],
                             buf0.at[pl.ds(0, 8), pl.ds(128 * c, 128)],
                             sem_i0)

        @pl.when(sB + r > 8)
        def _():
            for c in range(2):
                pltpu.async_copy(feat_hbm.at[gB + 1, c],
                                 buf0.at[pl.ds(8, 8), pl.ds(128 * c, 128)],
                                 sem_i0)

        for c in range(2):
            pltpu.make_async_copy(feat_hbm.at[0, c],
                                  buf0.at[pl.ds(0, 8), pl.ds(128 * c, 128)],
                                  sem_i0).wait()

        @pl.when(sB + r > 8)
        def _():
            for c in range(2):
                pltpu.make_async_copy(
                    feat_hbm.at[0, c],
                    buf0.at[pl.ds(8, 8), pl.ds(128 * c, 128)], sem_i0).wait()

        zvec = jnp.zeros((_LANES,), jnp.float32)
        for c in range(2):
            for i in range(8):
                li = jnp.where(i < r, sB + i, 0)
                for k in range(128 // _LANES):
                    vec = buf0[li, pl.ds(128 * c + k * _LANES, _LANES)]
                    gstage[c, i, pl.ds(k * _LANES, _LANES)] = jnp.where(
                        i < r, vec, zvec)
        pltpu.sync_copy(gstage, out_hbm.at[b, g0w + n8 // 8])

    def _zero_drain(i, carry):
        pltpu.make_async_copy(zbuf, out_hbm.at[b, pl.ds(g0w, _ZG)],
                              sem_zw).wait()
        return carry

    lax.fori_loop(0, zfull, _zero_drain, 0)
    for gbit in (8, 4, 2, 1):
        @pl.when((zrem & gbit) != 0)
        def _(gbit=gbit):
            pltpu.make_async_copy(zbuf.at[pl.ds(0, gbit)],
                                  out_hbm.at[b, pl.ds(g0w, gbit)],
                                  sem_zw).wait()


def kernel(feat, cu_seqlens):
    cu_pad = jnp.zeros((32,), jnp.int32).at[:_B + 1].set(
        cu_seqlens.astype(jnp.int32))
    zblock = jnp.zeros((_ZG, 2, 8, 128), jnp.float32)
    feat5 = feat.reshape(_NGRP, 8, 2, 128).transpose(0, 2, 1, 3)
    fn = pl.kernel(
        _pad_body,
        mesh=plsc.VectorSubcoreMesh(core_axis_name="c", subcore_axis_name="s"),
        compiler_params=pltpu.CompilerParams(use_tc_tiling_on_sc=False),
        out_type=jax.ShapeDtypeStruct((_B, _MAX_LEN // 8, 2, 8, 128),
                                      jnp.float32),
        scratch_types=[
            pltpu.VMEM((32,), jnp.int32),
            pltpu.VMEM((_CHUNK + 8, _D), jnp.float32),
            pltpu.VMEM((_CHUNK + 8, _D), jnp.float32),
            pltpu.VMEM((_ZG, 2, 8, 128), jnp.float32),
            pltpu.VMEM((2, 8, 128), jnp.float32),
            pltpu.SemaphoreType.DMA,
            pltpu.SemaphoreType.DMA,
            pltpu.SemaphoreType.DMA,
            pltpu.SemaphoreType.DMA,
            pltpu.SemaphoreType.DMA,
            pltpu.SemaphoreType.DMA,
        ],
    )
    out5 = fn(feat5, cu_pad, zblock)
    return (out5.transpose(0, 1, 3, 2, 4)
            .reshape(_B, _MAX_LEN, _D))


# speedup vs baseline: 2.4836x; 1.0534x over previous
import jax
import jax.numpy as jnp
from jax import lax
from jax.experimental import pallas as pl
from jax.experimental.pallas import tpu as pltpu
from jax.experimental.pallas import tpu_sc as plsc

_B = 16
_MAX_LEN = 2048
_D = 256
_TOTAL = 16384
_NGRP = _TOTAL // 8
_NC = 2
_NS = 16
_NW = _NC * _NS
_SLAB = 256
_NSLAB = (_B * _MAX_LEN) // _SLAB
_REPS = _NSLAB // _NW
_SPB = _MAX_LEN // _SLAB
_GPS = _SLAB // 8
_CHUNK = 128
_CG = _CHUNK // 8
_BITS = (64, 32, 16, 8)
_ZG = 16
_LANES = 16


def _pad_body(feat_hbm, cu_hbm, zeros_hbm, out_hbm,
              cu_v, buf0, buf1, zbuf, gstage,
              sem_i0, sem_i1, sem_o0, sem_o1, sem_z, sem_zw):
    w = lax.axis_index("s") * _NC + lax.axis_index("c")

    pltpu.sync_copy(cu_hbm, cu_v)
    pltpu.async_copy(zeros_hbm, zbuf, sem_z).wait()

    def _slab_body(rep, carry):
        slab = w + _NW * rep
        b = slab // _SPB
        g0w = (slab % _SPB) * _GPS
        p0 = (slab % _SPB) * _SLAB

        cu_pair = cu_v[pl.ds(b, 16)]
        start = cu_pair[0] + p0
        n = jnp.clip(cu_pair[1] - start, 0, _SLAB)
        n8 = n & ~7
        r = n - n8
        s = start & 7
        nfull = n8 // _CHUNK

        def _read_span(o, ngrp, buf, sem):
            gf = (start + o) >> 3
            for gg in range(ngrp):
                for c in range(2):
                    pltpu.async_copy(
                        feat_hbm.at[gf + gg, c],
                        buf.at[pl.ds(8 * gg, 8), pl.ds(128 * c, 128)], sem)

            @pl.when(s != 0)
            def _():
                for c in range(2):
                    pltpu.async_copy(
                        feat_hbm.at[gf + ngrp, c],
                        buf.at[pl.ds(8 * ngrp, 8), pl.ds(128 * c, 128)], sem)

        def _wait_read(ngrp, buf, sem):
            for gg in range(ngrp):
                for c in range(2):
                    pltpu.make_async_copy(
                        feat_hbm.at[0, c],
                        buf.at[pl.ds(8 * gg, 8), pl.ds(128 * c, 128)],
                        sem).wait()

            @pl.when(s != 0)
            def _():
                for c in range(2):
                    pltpu.make_async_copy(
                        feat_hbm.at[0, c],
                        buf.at[pl.ds(8 * ngrp, 8), pl.ds(128 * c, 128)],
                        sem).wait()

        def _write_span(o, ngrp, buf, sem):
            for gg in range(ngrp):
                for c in range(2):
                    pltpu.async_copy(
                        buf.at[pl.ds(s + 8 * gg, 8), pl.ds(128 * c, 128)],
                        out_hbm.at[b, g0w + o // 8 + gg, c], sem)

        def _wait_write(ngrp, buf, sem):
            for gg in range(ngrp):
                for c in range(2):
                    pltpu.make_async_copy(
                        buf.at[pl.ds(s + 8 * gg, 8), pl.ds(128 * c, 128)],
                        out_hbm.at[b, g0w, c], sem).wait()

        @pl.when(nfull > 0)
        def _():
            _read_span(0, _CG, buf0, sem_i0)

        @pl.when(nfull > 1)
        def _():
            _read_span(_CHUNK, _CG, buf1, sem_i1)

        zg0 = n8 // 8 + jnp.where(r != 0, 1, 0)
        zg = _GPS - zg0
        zfull = zg // _ZG

        def _zero_fire(i, c2):
            pltpu.async_copy(
                zbuf, out_hbm.at[b, pl.ds(g0w + zg0 + i * _ZG, _ZG)], sem_zw)
            return c2

        lax.fori_loop(0, zfull, _zero_fire, 0)

        zoff = zg0 + zfull * _ZG
        zrem = zg - zfull * _ZG
        for gbit in (8, 4, 2, 1):
            hi_mask = (_ZG - 1) ^ (2 * gbit - 1)

            @pl.when((zrem & gbit) != 0)
            def _(gbit=gbit, hi_mask=hi_mask):
                o = zoff + (zrem & hi_mask)
                pltpu.async_copy(zbuf.at[pl.ds(0, gbit)],
                                 out_hbm.at[b, pl.ds(g0w + o, gbit)], sem_zw)

        @pl.when(nfull > 0)
        def _():
            _wait_read(_CG, buf0, sem_i0)
            _write_span(0, _CG, buf0, sem_o0)

        @pl.when(nfull > 1)
        def _():
            _wait_read(_CG, buf1, sem_i1)
            _write_span(_CHUNK, _CG, buf1, sem_o1)

        @pl.when(nfull > 0)
        def _():
            _wait_write(_CG, buf0, sem_o0)

        @pl.when(nfull > 1)
        def _():
            _wait_write(_CG, buf1, sem_o1)

        off = nfull * _CHUNK
        rem = n8 - off
        for bit in _BITS:
            gbit = bit // 8
            hi_mask = (_CHUNK - 1) ^ (2 * bit - 1)

            @pl.when((rem & bit) != 0)
            def _(bit=bit, gbit=gbit, hi_mask=hi_mask):
                o = off + (rem & hi_mask)
                _read_span(o, gbit, buf0, sem_i0)
                _wait_read(gbit, buf0, sem_i0)
                _write_span(o, gbit, buf0, sem_o0)
                _wait_write(gbit, buf0, sem_o0)

        @pl.when(r != 0)
        def _():
            gB = (start + n8) >> 3
            sB = (start + n8) & 7
            for c in range(2):
                pltpu.async_copy(feat_hbm.at[gB, c],
                                 buf0.at[pl.ds(0, 8), pl.ds(128 * c, 128)],
                                 sem_i0)

            @pl.when(sB + r > 8)
            def _():
                for c in range(2):
                    pltpu.async_copy(
                        feat_hbm.at[gB + 1, c],
                        buf0.at[pl.ds(8, 8), pl.ds(128 * c, 128)], sem_i0)

            for c in range(2):
                pltpu.make_async_copy(
                    feat_hbm.at[0, c],
                    buf0.at[pl.ds(0, 8), pl.ds(128 * c, 128)], sem_i0).wait()

            @pl.when(sB + r > 8)
            def _():
                for c in range(2):
                    pltpu.make_async_copy(
                        feat_hbm.at[0, c],
                        buf0.at[pl.ds(8, 8), pl.ds(128 * c, 128)],
                        sem_i0).wait()

            zvec = jnp.zeros((_LANES,), jnp.float32)
            for c in range(2):
                for i in range(8):
                    li = jnp.where(i < r, sB + i, 0)
                    for k in range(128 // _LANES):
                        vec = buf0[li, pl.ds(128 * c + k * _LANES, _LANES)]
                        gstage[c, i, pl.ds(k * _LANES, _LANES)] = jnp.where(
                            i < r, vec, zvec)
            pltpu.sync_copy(gstage, out_hbm.at[b, g0w + n8 // 8])

        def _zero_drain(i, c2):
            pltpu.make_async_copy(zbuf, out_hbm.at[b, pl.ds(g0w, _ZG)],
                                  sem_zw).wait()
            return c2

        lax.fori_loop(0, zfull, _zero_drain, 0)
        for gbit in (8, 4, 2, 1):
            @pl.when((zrem & gbit) != 0)
            def _(gbit=gbit):
                pltpu.make_async_copy(zbuf.at[pl.ds(0, gbit)],
                                      out_hbm.at[b, pl.ds(g0w, gbit)],
                                      sem_zw).wait()

        return carry

    lax.fori_loop(0, _REPS, _slab_body, 0)


def kernel(feat, cu_seqlens):
    cu_pad = jnp.zeros((32,), jnp.int32).at[:_B + 1].set(
        cu_seqlens.astype(jnp.int32))
    zblock = jnp.zeros((_ZG, 2, 8, 128), jnp.float32)
    feat5 = feat.reshape(_NGRP, 8, 2, 128).transpose(0, 2, 1, 3)
    fn = pl.kernel(
        _pad_body,
        mesh=plsc.VectorSubcoreMesh(core_axis_name="c", subcore_axis_name="s"),
        compiler_params=pltpu.CompilerParams(use_tc_tiling_on_sc=False),
        out_type=jax.ShapeDtypeStruct((_B, _MAX_LEN // 8, 2, 8, 128),
                                      jnp.float32),
        scratch_types=[
            pltpu.VMEM((32,), jnp.int32),
            pltpu.VMEM((_CHUNK + 8, _D), jnp.float32),
            pltpu.VMEM((_CHUNK + 8, _D), jnp.float32),
            pltpu.VMEM((_ZG, 2, 8, 128), jnp.float32),
            pltpu.VMEM((2, 8, 128), jnp.float32),
            pltpu.SemaphoreType.DMA,
            pltpu.SemaphoreType.DMA,
            pltpu.SemaphoreType.DMA,
            pltpu.SemaphoreType.DMA,
            pltpu.SemaphoreType.DMA,
            pltpu.SemaphoreType.DMA,
        ],
    )
    out5 = fn(feat5, cu_pad, zblock)
    return (out5.transpose(0, 1, 3, 2, 4)
            .reshape(_B, _MAX_LEN, _D))


# speedup vs baseline: 2.5767x; 1.0375x over previous
import jax
import jax.numpy as jnp
from jax import lax
from jax.experimental import pallas as pl
from jax.experimental.pallas import tpu as pltpu
from jax.experimental.pallas import tpu_sc as plsc

_B = 16
_MAX_LEN = 2048
_D = 256
_TOTAL = 16384
_NGRP = _TOTAL // 8
_NC = 2
_NS = 16
_NW = _NC * _NS
_SLAB = 256
_NSLAB = (_B * _MAX_LEN) // _SLAB
_REPS = _NSLAB // _NW
_SPB = _MAX_LEN // _SLAB
_GPS = _SLAB // 8
_CHUNK = 128
_CG = _CHUNK // 8
_BITS = (64, 32, 16, 8)
_ZG = 16
_LANES = 16


def _pad_body(feat_hbm, cu_hbm, zeros_hbm, out_hbm,
              cu_v, buf0, buf1, zbuf, gstage,
              sem_i0, sem_i1, sem_o0, sem_o1, sem_z, sem_zw):
    w = lax.axis_index("s") * _NC + lax.axis_index("c")

    pltpu.sync_copy(cu_hbm, cu_v)
    pltpu.async_copy(zeros_hbm, zbuf, sem_z).wait()

    def _slab_body(rep, carry):
        slab = w + _NW * rep
        b = slab // _SPB
        g0w = (slab % _SPB) * _GPS
        p0 = (slab % _SPB) * _SLAB

        cu_pair = cu_v[pl.ds(b, 16)]
        start = cu_pair[0] + p0
        n = jnp.clip(cu_pair[1] - start, 0, _SLAB)
        n8 = n & ~7
        r = n - n8
        s = start & 7
        nfull = n8 // _CHUNK

        def _read_span(o, ngrp, buf, sem):
            gf = (start + o) >> 3
            for gg in range(ngrp):
                for c in range(2):
                    pltpu.async_copy(
                        feat_hbm.at[gf + gg, c],
                        buf.at[pl.ds(8 * gg, 8), pl.ds(128 * c, 128)], sem)

            @pl.when(s != 0)
            def _():
                for c in range(2):
                    pltpu.async_copy(
                        feat_hbm.at[gf + ngrp, c],
                        buf.at[pl.ds(8 * ngrp, 8), pl.ds(128 * c, 128)], sem)

        def _wait_read(ngrp, buf, sem):
            pltpu.make_async_copy(feat_hbm.at[pl.ds(0, ngrp)],
                                  zbuf.at[pl.ds(0, ngrp)], sem).wait()

            @pl.when(s != 0)
            def _():
                pltpu.make_async_copy(feat_hbm.at[pl.ds(0, 1)],
                                      zbuf.at[pl.ds(0, 1)], sem).wait()

        def _write_span(o, ngrp, buf, sem):
            for gg in range(ngrp):
                for c in range(2):
                    pltpu.async_copy(
                        buf.at[pl.ds(s + 8 * gg, 8), pl.ds(128 * c, 128)],
                        out_hbm.at[b, g0w + o // 8 + gg, c], sem)

        def _wait_write(ngrp, buf, sem):
            pltpu.make_async_copy(zbuf.at[pl.ds(0, ngrp)],
                                  out_hbm.at[0, pl.ds(0, ngrp)], sem).wait()

        @pl.when(nfull > 0)
        def _():
            _read_span(0, _CG, buf0, sem_i0)

        @pl.when(nfull > 1)
        def _():
            _read_span(_CHUNK, _CG, buf1, sem_i1)

        zg0 = n8 // 8 + jnp.where(r != 0, 1, 0)
        zg = _GPS - zg0
        zfull = zg // _ZG

        def _zero_fire(i, c2):
            pltpu.async_copy(
                zbuf, out_hbm.at[b, pl.ds(g0w + zg0 + i * _ZG, _ZG)], sem_zw)
            return c2

        lax.fori_loop(0, zfull, _zero_fire, 0)

        zoff = zg0 + zfull * _ZG
        zrem = zg - zfull * _ZG
        for gbit in (8, 4, 2, 1):
            hi_mask = (_ZG - 1) ^ (2 * gbit - 1)

            @pl.when((zrem & gbit) != 0)
            def _(gbit=gbit, hi_mask=hi_mask):
                o = zoff + (zrem & hi_mask)
                pltpu.async_copy(zbuf.at[pl.ds(0, gbit)],
                                 out_hbm.at[b, pl.ds(g0w + o, gbit)], sem_zw)

        @pl.when(nfull > 0)
        def _():
            _wait_read(_CG, buf0, sem_i0)
            _write_span(0, _CG, buf0, sem_o0)

        @pl.when(nfull > 1)
        def _():
            _wait_read(_CG, buf1, sem_i1)
            _write_span(_CHUNK, _CG, buf1, sem_o1)

        @pl.when(nfull > 0)
        def _():
            _wait_write(_CG, buf0, sem_o0)

        @pl.when(nfull > 1)
        def _():
            _wait_write(_CG, buf1, sem_o1)

        off = nfull * _CHUNK
        rem = n8 - off
        for bit in _BITS:
            gbit = bit // 8
            hi_mask = (_CHUNK - 1) ^ (2 * bit - 1)

            @pl.when((rem & bit) != 0)
            def _(bit=bit, gbit=gbit, hi_mask=hi_mask):
                o = off + (rem & hi_mask)
                _read_span(o, gbit, buf0, sem_i0)
                _wait_read(gbit, buf0, sem_i0)
                _write_span(o, gbit, buf0, sem_o0)
                _wait_write(gbit, buf0, sem_o0)

        @pl.when(r != 0)
        def _():
            gB = (start + n8) >> 3
            sB = (start + n8) & 7
            for c in range(2):
                pltpu.async_copy(feat_hbm.at[gB, c],
                                 buf0.at[pl.ds(0, 8), pl.ds(128 * c, 128)],
                                 sem_i0)

            @pl.when(sB + r > 8)
            def _():
                for c in range(2):
                    pltpu.async_copy(
                        feat_hbm.at[gB + 1, c],
                        buf0.at[pl.ds(8, 8), pl.ds(128 * c, 128)], sem_i0)

            pltpu.make_async_copy(feat_hbm.at[pl.ds(0, 1)],
                                  zbuf.at[pl.ds(0, 1)], sem_i0).wait()

            @pl.when(sB + r > 8)
            def _():
                pltpu.make_async_copy(feat_hbm.at[pl.ds(0, 1)],
                                      zbuf.at[pl.ds(0, 1)], sem_i0).wait()

            zvec = jnp.zeros((_LANES,), jnp.float32)
            for c in range(2):
                for i in range(8):
                    li = jnp.where(i < r, sB + i, 0)
                    for k in range(128 // _LANES):
                        vec = buf0[li, pl.ds(128 * c + k * _LANES, _LANES)]
                        gstage[c, i, pl.ds(k * _LANES, _LANES)] = jnp.where(
                            i < r, vec, zvec)
            pltpu.sync_copy(gstage, out_hbm.at[b, g0w + n8 // 8])

        def _zero_drain(i, c2):
            pltpu.make_async_copy(zbuf, out_hbm.at[b, pl.ds(g0w, _ZG)],
                                  sem_zw).wait()
            return c2

        lax.fori_loop(0, zfull, _zero_drain, 0)
        for gbit in (8, 4, 2, 1):
            @pl.when((zrem & gbit) != 0)
            def _(gbit=gbit):
                pltpu.make_async_copy(zbuf.at[pl.ds(0, gbit)],
                                      out_hbm.at[b, pl.ds(g0w, gbit)],
                                      sem_zw).wait()

        return carry

    lax.fori_loop(0, _REPS, _slab_body, 0)


def kernel(feat, cu_seqlens):
    cu_pad = jnp.zeros((32,), jnp.int32).at[:_B + 1].set(
        cu_seqlens.astype(jnp.int32))
    zblock = jnp.zeros((_ZG, 2, 8, 128), jnp.float32)
    feat5 = feat.reshape(_NGRP, 8, 2, 128).transpose(0, 2, 1, 3)
    fn = pl.kernel(
        _pad_body,
        mesh=plsc.VectorSubcoreMesh(core_axis_name="c", subcore_axis_name="s"),
        compiler_params=pltpu.CompilerParams(use_tc_tiling_on_sc=False),
        out_type=jax.ShapeDtypeStruct((_B, _MAX_LEN // 8, 2, 8, 128),
                                      jnp.float32),
        scratch_types=[
            pltpu.VMEM((32,), jnp.int32),
            pltpu.VMEM((_CHUNK + 8, _D), jnp.float32),
            pltpu.VMEM((_CHUNK + 8, _D), jnp.float32),
            pltpu.VMEM((_ZG, 2, 8, 128), jnp.float32),
            pltpu.VMEM((2, 8, 128), jnp.float32),
            pltpu.SemaphoreType.DMA,
            pltpu.SemaphoreType.DMA,
            pltpu.SemaphoreType.DMA,
            pltpu.SemaphoreType.DMA,
            pltpu.SemaphoreType.DMA,
            pltpu.SemaphoreType.DMA,
        ],
    )
    out5 = fn(feat5, cu_pad, zblock)
    return (out5.transpose(0, 1, 3, 2, 4)
            .reshape(_B, _MAX_LEN, _D))


# speedup vs baseline: 2.8203x; 1.0946x over previous
import jax
import jax.numpy as jnp
from jax import lax
from jax.experimental import pallas as pl
from jax.experimental.pallas import tpu as pltpu
from jax.experimental.pallas import tpu_sc as plsc

_B = 16
_MAX_LEN = 2048
_D = 256
_TOTAL = 16384
_NGRP = _TOTAL // 8
_NC = 2
_NS = 16
_NW = _NC * _NS
_SLAB = 256
_NSLAB = (_B * _MAX_LEN) // _SLAB
_REPS = _NSLAB // _NW
_SPB = _MAX_LEN // _SLAB
_GPS = _SLAB // 8
_CHUNK = 128
_CG = _CHUNK // 8
_BITS = (64, 32, 16, 8)
_ZG = 16
_LANES = 16


def _pad_body(feat_hbm, cu_hbm, zeros_hbm, out_hbm,
              cu_v, buf0, buf1, zbuf, gstage,
              sem_i0, sem_i1, sem_o0, sem_o1, sem_z, sem_zw):
    w = lax.axis_index("s") * _NC + lax.axis_index("c")

    pltpu.sync_copy(cu_hbm, cu_v.at[pl.ds(0, _B + 1)])
    pltpu.async_copy(zeros_hbm, zbuf, sem_z).wait()

    def _slab_body(rep, carry):
        slab = w + _NW * rep
        b = slab // _SPB
        g0w = (slab % _SPB) * _GPS
        p0 = (slab % _SPB) * _SLAB

        cu_pair = cu_v[pl.ds(b, 16)]
        start = cu_pair[0] + p0
        n = jnp.clip(cu_pair[1] - start, 0, _SLAB)
        n8 = n & ~7
        r = n - n8
        s = start & 7
        nfull = n8 // _CHUNK

        def _read_span(o, ngrp, buf, sem):
            gf = (start + o) >> 3

            def _one(gg, c2):
                for c in range(2):
                    pltpu.async_copy(
                        feat_hbm.at[gf + gg, c],
                        buf.at[pl.ds(8 * gg, 8), pl.ds(128 * c, 128)], sem)
                return c2

            lax.fori_loop(0, ngrp, _one, 0)

            @pl.when(s != 0)
            def _():
                for c in range(2):
                    pltpu.async_copy(
                        feat_hbm.at[gf + ngrp, c],
                        buf.at[pl.ds(8 * ngrp, 8), pl.ds(128 * c, 128)], sem)

        def _wait_read(ngrp, buf, sem):
            pltpu.make_async_copy(feat_hbm.at[pl.ds(0, ngrp)],
                                  zbuf.at[pl.ds(0, ngrp)], sem).wait()

            @pl.when(s != 0)
            def _():
                pltpu.make_async_copy(feat_hbm.at[pl.ds(0, 1)],
                                      zbuf.at[pl.ds(0, 1)], sem).wait()

        def _write_span(o, ngrp, buf, sem):
            def _one(gg, c2):
                for c in range(2):
                    pltpu.async_copy(
                        buf.at[pl.ds(s + 8 * gg, 8), pl.ds(128 * c, 128)],
                        out_hbm.at[b, g0w + o // 8 + gg, c], sem)
                return c2

            lax.fori_loop(0, ngrp, _one, 0)

        def _wait_write(ngrp, buf, sem):
            pltpu.make_async_copy(zbuf.at[pl.ds(0, ngrp)],
                                  out_hbm.at[0, pl.ds(0, ngrp)], sem).wait()

        @pl.when(nfull > 0)
        def _():
            _read_span(0, _CG, buf0, sem_i0)

        @pl.when(nfull > 1)
        def _():
            _read_span(_CHUNK, _CG, buf1, sem_i1)

        zg0 = n8 // 8 + jnp.where(r != 0, 1, 0)
        zg = _GPS - zg0
        zfull = zg // _ZG

        def _zero_fire(i, c2):
            pltpu.async_copy(
                zbuf, out_hbm.at[b, pl.ds(g0w + zg0 + i * _ZG, _ZG)], sem_zw)
            return c2

        lax.fori_loop(0, zfull, _zero_fire, 0)

        zoff = zg0 + zfull * _ZG
        zrem = zg - zfull * _ZG
        for gbit in (8, 4, 2, 1):
            hi_mask = (_ZG - 1) ^ (2 * gbit - 1)

            @pl.when((zrem & gbit) != 0)
            def _(gbit=gbit, hi_mask=hi_mask):
                o = zoff + (zrem & hi_mask)
                pltpu.async_copy(zbuf.at[pl.ds(0, gbit)],
                                 out_hbm.at[b, pl.ds(g0w + o, gbit)], sem_zw)

        @pl.when(nfull > 0)
        def _():
            _wait_read(_CG, buf0, sem_i0)
            _write_span(0, _CG, buf0, sem_o0)

        @pl.when(nfull > 1)
        def _():
            _wait_read(_CG, buf1, sem_i1)
            _write_span(_CHUNK, _CG, buf1, sem_o1)

        @pl.when(nfull > 0)
        def _():
            _wait_write(_CG, buf0, sem_o0)

        @pl.when(nfull > 1)
        def _():
            _wait_write(_CG, buf1, sem_o1)

        off = nfull * _CHUNK
        rem = n8 - off
        for bit in _BITS:
            gbit = bit // 8
            hi_mask = (_CHUNK - 1) ^ (2 * bit - 1)

            @pl.when((rem & bit) != 0)
            def _(bit=bit, gbit=gbit, hi_mask=hi_mask):
                o = off + (rem & hi_mask)
                _read_span(o, gbit, buf0, sem_i0)
                _wait_read(gbit, buf0, sem_i0)
                _write_span(o, gbit, buf0, sem_o0)
                _wait_write(gbit, buf0, sem_o0)

        @pl.when(r != 0)
        def _():
            gB = (start + n8) >> 3
            sB = (start + n8) & 7
            for c in range(2):
                pltpu.async_copy(feat_hbm.at[gB, c],
                                 buf0.at[pl.ds(0, 8), pl.ds(128 * c, 128)],
                                 sem_i0)

            @pl.when(sB + r > 8)
            def _():
                for c in range(2):
                    pltpu.async_copy(
                        feat_hbm.at[gB + 1, c],
                        buf0.at[pl.ds(8, 8), pl.ds(128 * c, 128)], sem_i0)

            pltpu.make_async_copy(feat_hbm.at[pl.ds(0, 1)],
                                  zbuf.at[pl.ds(0, 1)], sem_i0).wait()

            @pl.when(sB + r > 8)
            def _():
                pltpu.make_async_copy(feat_hbm.at[pl.ds(0, 1)],
                                      zbuf.at[pl.ds(0, 1)], sem_i0).wait()

            zvec = jnp.zeros((_LANES,), jnp.float32)
            for c in range(2):
                for i in range(8):
                    li = jnp.where(i < r, sB + i, 0)
                    for k in range(128 // _LANES):
                        vec = buf0[li, pl.ds(128 * c + k * _LANES, _LANES)]
                        gstage[c, i, pl.ds(k * _LANES, _LANES)] = jnp.where(
                            i < r, vec, zvec)
            pltpu.sync_copy(gstage, out_hbm.at[b, g0w + n8 // 8])

        def _zero_drain(i, c2):
            pltpu.make_async_copy(zbuf, out_hbm.at[b, pl.ds(g0w, _ZG)],
                                  sem_zw).wait()
            return c2

        lax.fori_loop(0, zfull, _zero_drain, 0)
        for gbit in (8, 4, 2, 1):
            @pl.when((zrem & gbit) != 0)
            def _(gbit=gbit):
                pltpu.make_async_copy(zbuf.at[pl.ds(0, gbit)],
                                      out_hbm.at[b, pl.ds(g0w, gbit)],
                                      sem_zw).wait()

        return carry

    lax.fori_loop(0, _REPS, _slab_body, 0)


def kernel(feat, cu_seqlens):
    cu32 = cu_seqlens.astype(jnp.int32)
    zblock = jnp.zeros((_ZG, 2, 8, 128), jnp.float32)
    feat5 = feat.reshape(_NGRP, 8, 2, 128).transpose(0, 2, 1, 3)
    fn = pl.kernel(
        _pad_body,
        mesh=plsc.VectorSubcoreMesh(core_axis_name="c", subcore_axis_name="s"),
        compiler_params=pltpu.CompilerParams(use_tc_tiling_on_sc=False),
        out_type=jax.ShapeDtypeStruct((_B, _MAX_LEN // 8, 2, 8, 128),
                                      jnp.float32),
        scratch_types=[
            pltpu.VMEM((32,), jnp.int32),
            pltpu.VMEM((_CHUNK + 8, _D), jnp.float32),
            pltpu.VMEM((_CHUNK + 8, _D), jnp.float32),
            pltpu.VMEM((_ZG, 2, 8, 128), jnp.float32),
            pltpu.VMEM((2, 8, 128), jnp.float32),
            pltpu.SemaphoreType.DMA,
            pltpu.SemaphoreType.DMA,
            pltpu.SemaphoreType.DMA,
            pltpu.SemaphoreType.DMA,
            pltpu.SemaphoreType.DMA,
            pltpu.SemaphoreType.DMA,
        ],
    )
    out5 = fn(feat5, cu32, zblock)
    return (out5.transpose(0, 1, 3, 2, 4)
            .reshape(_B, _MAX_LEN, _D))


# speedup vs baseline: 2.8770x; 1.0201x over previous
import jax
import jax.numpy as jnp
from jax import lax
from jax.experimental import pallas as pl
from jax.experimental.pallas import tpu as pltpu
from jax.experimental.pallas import tpu_sc as plsc

_B = 16
_MAX_LEN = 2048
_D = 256
_TOTAL = 16384
_NGRP = _TOTAL // 8
_NC = 2
_NS = 16
_NW = _NC * _NS
_SLAB = 256
_NSLAB = (_B * _MAX_LEN) // _SLAB
_REPS = _NSLAB // _NW
_SPB = _MAX_LEN // _SLAB
_GPS = _SLAB // 8
_CHUNK = 128
_CG = _CHUNK // 8
_BITS = (64, 32, 16, 8)
_ZG = 16
_LANES = 16


def _pad_body(feat_hbm, cu_hbm, zeros_hbm, out_hbm,
              cu_v, buf0, buf1, zbuf, gstage,
              sem_i0, sem_i1, sem_o0, sem_o1, sem_z, sem_zw):
    w = lax.axis_index("s") * _NC + lax.axis_index("c")

    zh = pltpu.async_copy(zeros_hbm, zbuf, sem_z)
    pltpu.sync_copy(cu_hbm, cu_v.at[pl.ds(0, _B + 1)])
    zh.wait()

    def _slab_body(rep, carry):
        slab = w + _NW * rep
        b = slab // _SPB
        g0w = (slab % _SPB) * _GPS
        p0 = (slab % _SPB) * _SLAB

        cu_pair = cu_v[pl.ds(b, 16)]
        start = cu_pair[0] + p0
        n = jnp.clip(cu_pair[1] - start, 0, _SLAB)
        n8 = n & ~7
        r = n - n8
        s = start & 7
        nfull = n8 // _CHUNK

        def _read_span(o, ngrp, buf, sem):
            gf = (start + o) >> 3

            def _one(gg, c2):
                for c in range(2):
                    pltpu.async_copy(
                        feat_hbm.at[gf + gg, c],
                        buf.at[pl.ds(8 * gg, 8), pl.ds(128 * c, 128)], sem)
                return c2

            lax.fori_loop(0, ngrp, _one, 0)

            @pl.when(s != 0)
            def _():
                for c in range(2):
                    pltpu.async_copy(
                        feat_hbm.at[gf + ngrp, c],
                        buf.at[pl.ds(8 * ngrp, 8), pl.ds(128 * c, 128)], sem)

        def _wait_read(ngrp, buf, sem):
            pltpu.make_async_copy(feat_hbm.at[pl.ds(0, ngrp)],
                                  zbuf.at[pl.ds(0, ngrp)], sem).wait()

            @pl.when(s != 0)
            def _():
                pltpu.make_async_copy(feat_hbm.at[pl.ds(0, 1)],
                                      zbuf.at[pl.ds(0, 1)], sem).wait()

        def _write_span(o, ngrp, buf, sem):
            def _one(gg, c2):
                for c in range(2):
                    pltpu.async_copy(
                        buf.at[pl.ds(s + 8 * gg, 8), pl.ds(128 * c, 128)],
                        out_hbm.at[b, g0w + o // 8 + gg, c], sem)
                return c2

            lax.fori_loop(0, ngrp, _one, 0)

        def _wait_write(ngrp, buf, sem):
            pltpu.make_async_copy(zbuf.at[pl.ds(0, ngrp)],
                                  out_hbm.at[0, pl.ds(0, ngrp)], sem).wait()

        @pl.when(nfull > 0)
        def _():
            _read_span(0, _CG, buf0, sem_i0)

        @pl.when(nfull > 1)
        def _():
            _read_span(_CHUNK, _CG, buf1, sem_i1)

        zg0 = n8 // 8 + jnp.where(r != 0, 1, 0)
        zg = _GPS - zg0
        zfull = zg // _ZG

        def _zero_fire(i, c2):
            pltpu.async_copy(
                zbuf, out_hbm.at[b, pl.ds(g0w + zg0 + i * _ZG, _ZG)], sem_zw)
            return c2

        lax.fori_loop(0, zfull, _zero_fire, 0)

        zoff = zg0 + zfull * _ZG
        zrem = zg - zfull * _ZG
        for gbit in (8, 4, 2, 1):
            hi_mask = (_ZG - 1) ^ (2 * gbit - 1)

            @pl.when((zrem & gbit) != 0)
            def _(gbit=gbit, hi_mask=hi_mask):
                o = zoff + (zrem & hi_mask)
                pltpu.async_copy(zbuf.at[pl.ds(0, gbit)],
                                 out_hbm.at[b, pl.ds(g0w + o, gbit)], sem_zw)

        @pl.when(nfull > 0)
        def _():
            _wait_read(_CG, buf0, sem_i0)
            _write_span(0, _CG, buf0, sem_o0)

        @pl.when(nfull > 1)
        def _():
            _wait_read(_CG, buf1, sem_i1)
            _write_span(_CHUNK, _CG, buf1, sem_o1)

        @pl.when(nfull > 0)
        def _():
            _wait_write(_CG, buf0, sem_o0)

        @pl.when(nfull > 1)
        def _():
            _wait_write(_CG, buf1, sem_o1)

        off = nfull * _CHUNK
        rem = n8 - off
        for bit in _BITS:
            gbit = bit // 8
            hi_mask = (_CHUNK - 1) ^ (2 * bit - 1)

            @pl.when((rem & bit) != 0)
            def _(bit=bit, gbit=gbit, hi_mask=hi_mask):
                o = off + (rem & hi_mask)
                _read_span(o, gbit, buf0, sem_i0)
                _wait_read(gbit, buf0, sem_i0)
                _write_span(o, gbit, buf0, sem_o0)
                _wait_write(gbit, buf0, sem_o0)

        @pl.when(r != 0)
        def _():
            gB = (start + n8) >> 3
            sB = (start + n8) & 7
            for c in range(2):
                pltpu.async_copy(feat_hbm.at[gB, c],
                                 buf0.at[pl.ds(0, 8), pl.ds(128 * c, 128)],
                                 sem_i0)

            @pl.when(sB + r > 8)
            def _():
                for c in range(2):
                    pltpu.async_copy(
                        feat_hbm.at[gB + 1, c],
                        buf0.at[pl.ds(8, 8), pl.ds(128 * c, 128)], sem_i0)

            pltpu.make_async_copy(feat_hbm.at[pl.ds(0, 1)],
                                  zbuf.at[pl.ds(0, 1)], sem_i0).wait()

            @pl.when(sB + r > 8)
            def _():
                pltpu.make_async_copy(feat_hbm.at[pl.ds(0, 1)],
                                      zbuf.at[pl.ds(0, 1)], sem_i0).wait()

            zvec = jnp.zeros((_LANES,), jnp.float32)
            for c in range(2):
                for i in range(8):
                    li = jnp.where(i < r, sB + i, 0)
                    for k in range(128 // _LANES):
                        vec = buf0[li, pl.ds(128 * c + k * _LANES, _LANES)]
                        gstage[c, i, pl.ds(k * _LANES, _LANES)] = jnp.where(
                            i < r, vec, zvec)
            pltpu.sync_copy(gstage, out_hbm.at[b, g0w + n8 // 8])

        def _zero_drain(i, c2):
            pltpu.make_async_copy(zbuf, out_hbm.at[b, pl.ds(g0w, _ZG)],
                                  sem_zw).wait()
            return c2

        lax.fori_loop(0, zfull, _zero_drain, 0)
        for gbit in (8, 4, 2, 1):
            @pl.when((zrem & gbit) != 0)
            def _(gbit=gbit):
                pltpu.make_async_copy(zbuf.at[pl.ds(0, gbit)],
                                      out_hbm.at[b, pl.ds(g0w, gbit)],
                                      sem_zw).wait()

        return carry

    lax.fori_loop(0, _REPS, _slab_body, 0)


def kernel(feat, cu_seqlens):
    cu32 = cu_seqlens.astype(jnp.int32)
    zblock = jnp.zeros((_ZG, 2, 8, 128), jnp.float32)
    feat5 = feat.reshape(_NGRP, 8, 2, 128).transpose(0, 2, 1, 3)
    fn = pl.kernel(
        _pad_body,
        mesh=plsc.VectorSubcoreMesh(core_axis_name="c", subcore_axis_name="s"),
        compiler_params=pltpu.CompilerParams(use_tc_tiling_on_sc=False),
        out_type=jax.ShapeDtypeStruct((_B, _MAX_LEN // 8, 2, 8, 128),
                                      jnp.float32),
        scratch_types=[
            pltpu.VMEM((32,), jnp.int32),
            pltpu.VMEM((_CHUNK + 8, _D), jnp.float32),
            pltpu.VMEM((_CHUNK + 8, _D), jnp.float32),
            pltpu.VMEM((_ZG, 2, 8, 128), jnp.float32),
            pltpu.VMEM((2, 8, 128), jnp.float32),
            pltpu.SemaphoreType.DMA,
            pltpu.SemaphoreType.DMA,
            pltpu.SemaphoreType.DMA,
            pltpu.SemaphoreType.DMA,
            pltpu.SemaphoreType.DMA,
            pltpu.SemaphoreType.DMA,
        ],
    )
    out5 = fn(feat5, cu32, zblock)
    return (out5.transpose(0, 1, 3, 2, 4)
            .reshape(_B, _MAX_LEN, _D))


# speedup vs baseline: 2.8972x; 1.0070x over previous
import jax
import jax.numpy as jnp
from jax import lax
from jax.experimental import pallas as pl
from jax.experimental.pallas import tpu as pltpu
from jax.experimental.pallas import tpu_sc as plsc

_B = 16
_MAX_LEN = 2048
_D = 256
_TOTAL = 16384
_NGRP = _TOTAL // 8
_NC = 2
_NS = 16
_NW = _NC * _NS
_SLAB = 256
_NSLAB = (_B * _MAX_LEN) // _SLAB
_REPS = _NSLAB // _NW
_SPB = _MAX_LEN // _SLAB
_GPS = _SLAB // 8
_CHUNK = 128
_CG = _CHUNK // 8
_BITS = (64, 32, 16, 8)
_ZG = 16
_LANES = 16


def _pad_body(feat_hbm, cu_hbm, zeros_hbm, out_hbm,
              cu_v, buf0, buf1, zbuf, gstage,
              sem_i0, sem_i1, sem_o0, sem_o1, sem_z, sem_zw):
    w = lax.axis_index("s") * _NC + lax.axis_index("c")

    zh = pltpu.async_copy(zeros_hbm, zbuf, sem_z)
    pltpu.sync_copy(cu_hbm, cu_v.at[pl.ds(0, _B + 1)])
    zh.wait()

    def _slab_body(rep, carry):
        slab = w + _NW * rep
        b = slab // _SPB
        g0w = (slab % _SPB) * _GPS
        p0 = (slab % _SPB) * _SLAB

        cu_pair = cu_v[pl.ds(b, 16)]
        start = cu_pair[0] + p0
        n = jnp.clip(cu_pair[1] - start, 0, _SLAB)
        n8 = n & ~7
        r = n - n8
        s = start & 7
        nfull = n8 // _CHUNK

        def _read_span(o, ngrp, buf, sem):
            gf = (start + o) >> 3

            def _one(gg, c2):
                for c in range(2):
                    pltpu.async_copy(
                        feat_hbm.at[gf + gg, c],
                        buf.at[pl.ds(8 * gg, 8), pl.ds(128 * c, 128)], sem)
                return c2

            lax.fori_loop(0, ngrp, _one, 0)

            @pl.when(s != 0)
            def _():
                for c in range(2):
                    pltpu.async_copy(
                        feat_hbm.at[gf + ngrp, c],
                        buf.at[pl.ds(8 * ngrp, 8), pl.ds(128 * c, 128)], sem)

        def _wait_read(ngrp, buf, sem):
            pltpu.make_async_copy(feat_hbm.at[pl.ds(0, ngrp)],
                                  zbuf.at[pl.ds(0, ngrp)], sem).wait()

            @pl.when(s != 0)
            def _():
                pltpu.make_async_copy(feat_hbm.at[pl.ds(0, 1)],
                                      zbuf.at[pl.ds(0, 1)], sem).wait()

        def _write_span(o, ngrp, buf, sem):
            def _one(gg, c2):
                for c in range(2):
                    pltpu.async_copy(
                        buf.at[pl.ds(s + 8 * gg, 8), pl.ds(128 * c, 128)],
                        out_hbm.at[b, g0w + o // 8 + gg, c], sem)
                return c2

            lax.fori_loop(0, ngrp, _one, 0)

        def _wait_write(ngrp, buf, sem):
            pltpu.make_async_copy(zbuf.at[pl.ds(0, ngrp)],
                                  out_hbm.at[0, pl.ds(0, ngrp)], sem).wait()

        @pl.when(nfull > 0)
        def _():
            _read_span(0, _CG, buf0, sem_i0)

        @pl.when(nfull > 1)
        def _():
            _read_span(_CHUNK, _CG, buf1, sem_i1)

        zg0 = n8 // 8 + jnp.where(r != 0, 1, 0)
        zg = _GPS - zg0
        zfull = zg // _ZG

        def _zero_fire(i, c2):
            pltpu.async_copy(
                zbuf, out_hbm.at[b, pl.ds(g0w + zg0 + i * _ZG, _ZG)], sem_zw)
            return c2

        lax.fori_loop(0, zfull, _zero_fire, 0)

        zoff = zg0 + zfull * _ZG
        zrem = zg - zfull * _ZG
        for gbit in (8, 4, 2, 1):
            hi_mask = (_ZG - 1) ^ (2 * gbit - 1)

            @pl.when((zrem & gbit) != 0)
            def _(gbit=gbit, hi_mask=hi_mask):
                o = zoff + (zrem & hi_mask)
                pltpu.async_copy(zbuf.at[pl.ds(0, gbit)],
                                 out_hbm.at[b, pl.ds(g0w + o, gbit)], sem_zw)

        @pl.when(nfull > 0)
        def _():
            _wait_read(_CG, buf0, sem_i0)
            _write_span(0, _CG, buf0, sem_o0)

        @pl.when(nfull > 1)
        def _():
            _wait_read(_CG, buf1, sem_i1)
            _write_span(_CHUNK, _CG, buf1, sem_o1)

        @pl.when(nfull > 0)
        def _():
            _wait_write(_CG, buf0, sem_o0)

        off = nfull * _CHUNK
        rem = n8 - off
        for bit in _BITS:
            gbit = bit // 8
            hi_mask = (_CHUNK - 1) ^ (2 * bit - 1)

            @pl.when((rem & bit) != 0)
            def _(bit=bit, gbit=gbit, hi_mask=hi_mask):
                o = off + (rem & hi_mask)
                _read_span(o, gbit, buf0, sem_i0)
                _wait_read(gbit, buf0, sem_i0)
                _write_span(o, gbit, buf0, sem_o0)
                _wait_write(gbit, buf0, sem_o0)

        @pl.when(r != 0)
        def _():
            gB = (start + n8) >> 3
            sB = (start + n8) & 7
            for c in range(2):
                pltpu.async_copy(feat_hbm.at[gB, c],
                                 buf0.at[pl.ds(0, 8), pl.ds(128 * c, 128)],
                                 sem_i0)

            @pl.when(sB + r > 8)
            def _():
                for c in range(2):
                    pltpu.async_copy(
                        feat_hbm.at[gB + 1, c],
                        buf0.at[pl.ds(8, 8), pl.ds(128 * c, 128)], sem_i0)

            pltpu.make_async_copy(feat_hbm.at[pl.ds(0, 1)],
                                  zbuf.at[pl.ds(0, 1)], sem_i0).wait()

            @pl.when(sB + r > 8)
            def _():
                pltpu.make_async_copy(feat_hbm.at[pl.ds(0, 1)],
                                      zbuf.at[pl.ds(0, 1)], sem_i0).wait()

            zvec = jnp.zeros((_LANES,), jnp.float32)
            for c in range(2):
                for i in range(8):
                    li = jnp.where(i < r, sB + i, 0)
                    for k in range(128 // _LANES):
                        vec = buf0[li, pl.ds(128 * c + k * _LANES, _LANES)]
                        gstage[c, i, pl.ds(k * _LANES, _LANES)] = jnp.where(
                            i < r, vec, zvec)
            pltpu.sync_copy(gstage, out_hbm.at[b, g0w + n8 // 8])

        @pl.when(nfull > 1)
        def _():
            _wait_write(_CG, buf1, sem_o1)

        return carry + zg

    zg_total = lax.fori_loop(0, _REPS, _slab_body, 0)

    def _zero_drain(i, c2):
        pltpu.make_async_copy(zbuf, out_hbm.at[0, pl.ds(0, _ZG)],
                              sem_zw).wait()
        return c2

    lax.fori_loop(0, zg_total // _ZG, _zero_drain, 0)
    zr = zg_total % _ZG
    for gbit in (8, 4, 2, 1):
        @pl.when((zr & gbit) != 0)
        def _(gbit=gbit):
            pltpu.make_async_copy(zbuf.at[pl.ds(0, gbit)],
                                  out_hbm.at[0, pl.ds(0, gbit)],
                                  sem_zw).wait()


def kernel(feat, cu_seqlens):
    cu32 = cu_seqlens.astype(jnp.int32)
    zblock = jnp.zeros((_ZG, 2, 8, 128), jnp.float32)
    feat5 = feat.reshape(_NGRP, 8, 2, 128).transpose(0, 2, 1, 3)
    fn = pl.kernel(
        _pad_body,
        mesh=plsc.VectorSubcoreMesh(core_axis_name="c", subcore_axis_name="s"),
        compiler_params=pltpu.CompilerParams(use_tc_tiling_on_sc=False),
        out_type=jax.ShapeDtypeStruct((_B, _MAX_LEN // 8, 2, 8, 128),
                                      jnp.float32),
        scratch_types=[
            pltpu.VMEM((32,), jnp.int32),
            pltpu.VMEM((_CHUNK + 8, _D), jnp.float32),
            pltpu.VMEM((_CHUNK + 8, _D), jnp.float32),
            pltpu.VMEM((_ZG, 2, 8, 128), jnp.float32),
            pltpu.VMEM((2, 8, 128), jnp.float32),
            pltpu.SemaphoreType.DMA,
            pltpu.SemaphoreType.DMA,
            pltpu.SemaphoreType.DMA,
            pltpu.SemaphoreType.DMA,
            pltpu.SemaphoreType.DMA,
            pltpu.SemaphoreType.DMA,
        ],
    )
    out5 = fn(feat5, cu32, zblock)
    return (out5.transpose(0, 1, 3, 2, 4)
            .reshape(_B, _MAX_LEN, _D))


# speedup vs baseline: 2.9214x; 1.0084x over previous
import jax
import jax.numpy as jnp
from jax import lax
from jax.experimental import pallas as pl
from jax.experimental.pallas import tpu as pltpu
from jax.experimental.pallas import tpu_sc as plsc

_B = 16
_MAX_LEN = 2048
_D = 256
_TOTAL = 16384
_NGRP = _TOTAL // 8
_NC = 2
_NS = 16
_NW = _NC * _NS
_SLAB = 256
_NSLAB = (_B * _MAX_LEN) // _SLAB
_REPS = _NSLAB // _NW
_SPB = _MAX_LEN // _SLAB
_GPS = _SLAB // 8
_CHUNK = 128
_CG = _CHUNK // 8
_BITS = (64, 32, 16, 8)
_ZG = 16
_LANES = 16


def _pad_body(feat_hbm, cu_hbm, zeros_hbm, out_hbm,
              cu_v, buf0, buf1, zbuf, gstage, bbuf,
              sem_i0, sem_i1, sem_o0, sem_o1, sem_z, sem_zw, sem_ib):
    w = lax.axis_index("s") * _NC + lax.axis_index("c")

    zh = pltpu.async_copy(zeros_hbm, zbuf, sem_z)
    pltpu.sync_copy(cu_hbm, cu_v.at[pl.ds(0, _B + 1)])
    zh.wait()

    def _slab_body(rep, carry):
        slab = w + _NW * rep
        b = slab // _SPB
        g0w = (slab % _SPB) * _GPS
        p0 = (slab % _SPB) * _SLAB

        cu_pair = cu_v[pl.ds(b, 16)]
        start = cu_pair[0] + p0
        n = jnp.clip(cu_pair[1] - start, 0, _SLAB)
        n8 = n & ~7
        r = n - n8
        s = start & 7
        nfull = n8 // _CHUNK

        def _read_span(o, ngrp, buf, sem):
            gf = (start + o) >> 3

            def _one(gg, c2):
                for c in range(2):
                    pltpu.async_copy(
                        feat_hbm.at[gf + gg, c],
                        buf.at[pl.ds(8 * gg, 8), pl.ds(128 * c, 128)], sem)
                return c2

            lax.fori_loop(0, ngrp, _one, 0)

            @pl.when(s != 0)
            def _():
                for c in range(2):
                    pltpu.async_copy(
                        feat_hbm.at[gf + ngrp, c],
                        buf.at[pl.ds(8 * ngrp, 8), pl.ds(128 * c, 128)], sem)

        def _wait_read(ngrp, buf, sem):
            pltpu.make_async_copy(feat_hbm.at[pl.ds(0, ngrp)],
                                  zbuf.at[pl.ds(0, ngrp)], sem).wait()

            @pl.when(s != 0)
            def _():
                pltpu.make_async_copy(feat_hbm.at[pl.ds(0, 1)],
                                      zbuf.at[pl.ds(0, 1)], sem).wait()

        def _write_span(o, ngrp, buf, sem):
            def _one(gg, c2):
                for c in range(2):
                    pltpu.async_copy(
                        buf.at[pl.ds(s + 8 * gg, 8), pl.ds(128 * c, 128)],
                        out_hbm.at[b, g0w + o // 8 + gg, c], sem)
                return c2

            lax.fori_loop(0, ngrp, _one, 0)

        def _wait_write(ngrp, buf, sem):
            pltpu.make_async_copy(zbuf.at[pl.ds(0, ngrp)],
                                  out_hbm.at[0, pl.ds(0, ngrp)], sem).wait()

        @pl.when(nfull > 0)
        def _():
            _read_span(0, _CG, buf0, sem_i0)

        @pl.when(nfull > 1)
        def _():
            _read_span(_CHUNK, _CG, buf1, sem_i1)

        gB = (start + n8) >> 3
        sB = (start + n8) & 7

        @pl.when(r != 0)
        def _():
            for c in range(2):
                pltpu.async_copy(feat_hbm.at[gB, c],
                                 bbuf.at[pl.ds(0, 8), pl.ds(128 * c, 128)],
                                 sem_ib)

            @pl.when(sB + r > 8)
            def _():
                for c in range(2):
                    pltpu.async_copy(
                        feat_hbm.at[gB + 1, c],
                        bbuf.at[pl.ds(8, 8), pl.ds(128 * c, 128)], sem_ib)

        zg0 = n8 // 8 + jnp.where(r != 0, 1, 0)
        zg = _GPS - zg0
        zfull = zg // _ZG

        def _zero_fire(i, c2):
            pltpu.async_copy(
                zbuf, out_hbm.at[b, pl.ds(g0w + zg0 + i * _ZG, _ZG)], sem_zw)
            return c2

        lax.fori_loop(0, zfull, _zero_fire, 0)

        zoff = zg0 + zfull * _ZG
        zrem = zg - zfull * _ZG
        for gbit in (8, 4, 2, 1):
            hi_mask = (_ZG - 1) ^ (2 * gbit - 1)

            @pl.when((zrem & gbit) != 0)
            def _(gbit=gbit, hi_mask=hi_mask):
                o = zoff + (zrem & hi_mask)
                pltpu.async_copy(zbuf.at[pl.ds(0, gbit)],
                                 out_hbm.at[b, pl.ds(g0w + o, gbit)], sem_zw)

        @pl.when(nfull > 0)
        def _():
            _wait_read(_CG, buf0, sem_i0)
            _write_span(0, _CG, buf0, sem_o0)

        @pl.when(nfull > 1)
        def _():
            _wait_read(_CG, buf1, sem_i1)
            _write_span(_CHUNK, _CG, buf1, sem_o1)

        @pl.when(nfull > 0)
        def _():
            _wait_write(_CG, buf0, sem_o0)

        off = nfull * _CHUNK
        rem = n8 - off
        for bit in _BITS:
            gbit = bit // 8
            hi_mask = (_CHUNK - 1) ^ (2 * bit - 1)

            @pl.when((rem & bit) != 0)
            def _(bit=bit, gbit=gbit, hi_mask=hi_mask):
                o = off + (rem & hi_mask)
                _read_span(o, gbit, buf0, sem_i0)
                _wait_read(gbit, buf0, sem_i0)
                _write_span(o, gbit, buf0, sem_o0)
                _wait_write(gbit, buf0, sem_o0)

        @pl.when(r != 0)
        def _():
            pltpu.make_async_copy(feat_hbm.at[pl.ds(0, 1)],
                                  zbuf.at[pl.ds(0, 1)], sem_ib).wait()

            @pl.when(sB + r > 8)
            def _():
                pltpu.make_async_copy(feat_hbm.at[pl.ds(0, 1)],
                                      zbuf.at[pl.ds(0, 1)], sem_ib).wait()

            zvec = jnp.zeros((_LANES,), jnp.float32)
            for c in range(2):
                for i in range(8):
                    li = jnp.where(i < r, sB + i, 0)
                    for k in range(128 // _LANES):
                        vec = bbuf[li, pl.ds(128 * c + k * _LANES, _LANES)]
                        gstage[c, i, pl.ds(k * _LANES, _LANES)] = jnp.where(
                            i < r, vec, zvec)
            pltpu.sync_copy(gstage, out_hbm.at[b, g0w + n8 // 8])

        @pl.when(nfull > 1)
        def _():
            _wait_write(_CG, buf1, sem_o1)

        return carry + zg

    zg_total = lax.fori_loop(0, _REPS, _slab_body, 0)

    def _zero_drain(i, c2):
        pltpu.make_async_copy(zbuf, out_hbm.at[0, pl.ds(0, _ZG)],
                              sem_zw).wait()
        return c2

    lax.fori_loop(0, zg_total // _ZG, _zero_drain, 0)
    zr = zg_total % _ZG
    for gbit in (8, 4, 2, 1):
        @pl.when((zr & gbit) != 0)
        def _(gbit=gbit):
            pltpu.make_async_copy(zbuf.at[pl.ds(0, gbit)],
                                  out_hbm.at[0, pl.ds(0, gbit)],
                                  sem_zw).wait()


def kernel(feat, cu_seqlens):
    cu32 = cu_seqlens.astype(jnp.int32)
    zblock = jnp.zeros((_ZG, 2, 8, 128), jnp.float32)
    feat5 = feat.reshape(_NGRP, 8, 2, 128).transpose(0, 2, 1, 3)
    fn = pl.kernel(
        _pad_body,
        mesh=plsc.VectorSubcoreMesh(core_axis_name="c", subcore_axis_name="s"),
        compiler_params=pltpu.CompilerParams(use_tc_tiling_on_sc=False),
        out_type=jax.ShapeDtypeStruct((_B, _MAX_LEN // 8, 2, 8, 128),
                                      jnp.float32),
        scratch_types=[
            pltpu.VMEM((32,), jnp.int32),
            pltpu.VMEM((_CHUNK + 8, _D), jnp.float32),
            pltpu.VMEM((_CHUNK + 8, _D), jnp.float32),
            pltpu.VMEM((_ZG, 2, 8, 128), jnp.float32),
            pltpu.VMEM((2, 8, 128), jnp.float32),
            pltpu.VMEM((16, _D), jnp.float32),
            pltpu.SemaphoreType.DMA,
            pltpu.SemaphoreType.DMA,
            pltpu.SemaphoreType.DMA,
            pltpu.SemaphoreType.DMA,
            pltpu.SemaphoreType.DMA,
            pltpu.SemaphoreType.DMA,
            pltpu.SemaphoreType.DMA,
        ],
    )
    out5 = fn(feat5, cu32, zblock)
    return (out5.transpose(0, 1, 3, 2, 4)
            .reshape(_B, _MAX_LEN, _D))
